# Initial kernel scaffold; baseline (speedup 1.0000x reference)
import jax
import jax.numpy as jnp
from jax.experimental import pallas as pl


def kernel(x):
    raise NotImplementedError("write your pallas kernel here")


# speedup vs baseline: 22.4066x; 22.4066x over previous
import jax
import jax.numpy as jnp
from jax.experimental import pallas as pl


def _colsum_sq_kernel(x_ref, o_ref):
    y = jnp.maximum(x_ref[...], 0.0)
    s = jnp.sum(y, axis=0, keepdims=True)
    o_ref[...] = jnp.sum(s * s, keepdims=True)


def kernel(x):
    out = pl.pallas_call(
        _colsum_sq_kernel,
        out_shape=jax.ShapeDtypeStruct((1, 1), jnp.float32),
    )(x)
    return out[0, 0]
